# baseline (speedup 1.0000x reference)
import functools

import jax
import jax.numpy as jnp
from jax import lax
from jax.experimental import pallas as pl
from jax.experimental.pallas import tpu as pltpu
from jax.experimental.pallas import tpu_sc as plsc

VOCAB = 1000000
EMB = 64
HID = 128
B = 4096
L = 200

NC = 2
NS = 16
NW = NC * NS
ROWS_TOTAL = 2 * B
ROWS_PER_W = ROWS_TOTAL // NW
CHUNKS = ((0, 128), (128, 72))


def _sc_body(idx_hbm, table_hbm, out_hbm, idx_v, buf, out_v, sem):
    wid = lax.axis_index("s") * NC + lax.axis_index("c")
    base = wid * ROWS_PER_W
    pltpu.sync_copy(idx_hbm.at[pl.ds(base * L, ROWS_PER_W * L)], idx_v)

    def accum_body(r, accs):
        return tuple(accs[c] + buf[r, pl.ds(c * 16, 16)] for c in range(4))

    def row_body(i, carry):
        off = pl.multiple_of(i * L, 8)
        copies = [
            pltpu.async_copy(
                table_hbm.at[idx_v.at[pl.ds(off + c0, n)]],
                buf.at[pl.ds(c0, n)],
                sem,
            )
            for (c0, n) in CHUNKS
        ]
        for c in copies:
            c.wait()
        zeros = tuple(jnp.zeros((16,), jnp.float32) for _ in range(4))
        accs = lax.fori_loop(0, L, accum_body, zeros)
        for c in range(4):
            out_v[i, pl.ds(c * 16, 16)] = accs[c] * (1.0 / L)
        return carry

    lax.fori_loop(0, ROWS_PER_W, row_body, 0)
    pltpu.sync_copy(out_v, out_hbm.at[pl.ds(base, ROWS_PER_W)])


@jax.jit
def _sc_gather_mean(idx_flat, table):
    mesh = plsc.VectorSubcoreMesh(
        core_axis_name="c", subcore_axis_name="s", num_cores=NC, num_subcores=NS
    )
    return pl.kernel(
        _sc_body,
        out_type=jax.ShapeDtypeStruct((ROWS_TOTAL, EMB), jnp.float32),
        mesh=mesh,
        scratch_types=[
            pltpu.VMEM((ROWS_PER_W * L,), jnp.int32),
            pltpu.VMEM((L, EMB), jnp.float32),
            pltpu.VMEM((ROWS_PER_W, EMB), jnp.float32),
            pltpu.SemaphoreType.DMA,
        ],
        compiler_params=pltpu.CompilerParams(use_tc_tiling_on_sc=False),
        name="sc_gather_mean",
    )(idx_flat, table)


def _mlp_body(xc_ref, xs_ref, w1_ref, b1_ref, w2_ref, b2_ref, o_ref):
    w1 = w1_ref[...]
    b1 = b1_ref[...]
    w2 = w2_ref[...]
    b2 = b2_ref[...]
    hc = jax.nn.relu(
        jnp.dot(xc_ref[...], w1, preferred_element_type=jnp.float32) + b1
    )
    hs = jax.nn.relu(
        jnp.dot(xs_ref[...], w1, preferred_element_type=jnp.float32) + b1
    )
    hc = jax.nn.relu(jnp.dot(hc, w2, preferred_element_type=jnp.float32) + b2)
    hs = jax.nn.relu(jnp.dot(hs, w2, preferred_element_type=jnp.float32) + b2)
    d = hc - hs
    o_ref[...] = jnp.sqrt(jnp.sum(d * d, axis=1))


@jax.jit
def _mlp_distance(xc, xs, w1t, b1, w2t, b2):
    return pl.pallas_call(
        _mlp_body,
        out_shape=jax.ShapeDtypeStruct((B,), jnp.float32),
    )(xc, xs, w1t, b1, w2t, b2)


def kernel(correct_sql, student_sql, table, fc_w, fc_b, out_w, out_b):
    idx_flat = jnp.concatenate(
        [correct_sql.astype(jnp.int32), student_sql.astype(jnp.int32)], axis=0
    ).reshape(-1)
    pooled = _sc_gather_mean(idx_flat, table)
    xc = pooled[:B]
    xs = pooled[B:]
    return _mlp_distance(
        xc, xs, fc_w.T, fc_b[None, :], out_w.T, out_b[None, :]
    )
